# baseline (speedup 1.0000x reference)
import jax
import jax.numpy as jnp
from jax.experimental import pallas as pl
from jax.experimental.pallas import tpu as pltpu

_V = 50000
_H = 1024
_L = 400
_B = 64
_PAD = 500
_NHI = 101
_NLO = 500
_WPAD = _NHI * _NLO
_TV = 4000
_NV = 13
_RPS = _TV // _NLO
_TL = 80
_NLB = 5
_RB = 2
_F32 = jnp.float32


def _dot_nt(x, w):
    return jax.lax.dot_general(
        x, w, (((1,), (1,)), ((), ())), preferred_element_type=_F32)


def _embed_attn_apply_body(ids_ref, emb_hbm, h0_ref, aW_ref, ab_ref, enc_ref,
                           eout_ref, w_ref, acc_ref, wT_ref, sem):
    j = pl.program_id(0)

    @pl.when(j == 0)
    def _():
        cps = [pltpu.make_async_copy(emb_hbm.at[pl.ds(ids_ref[i], 1), :],
                                     eout_ref.at[pl.ds(i, 1), :], sem)
               for i in range(_B)]
        for cp in cps:
            cp.start()
        for cp in cps:
            cp.wait()
        emb = eout_ref[...]
        logits = (_dot_nt(emb, aW_ref[:, :_H])
                  + _dot_nt(h0_ref[...], aW_ref[:, _H:]) + ab_ref[...])
        m = jnp.max(logits, axis=1, keepdims=True)
        e = jnp.exp(logits - m)
        w = e / jnp.sum(e, axis=1, keepdims=True)
        w_ref[...] = w
        wT_ref[...] = w.T

    off = pl.multiple_of(j * _TL, 8)
    wblk = jnp.transpose(wT_ref[pl.ds(off, _TL), :])
    part = wblk[:, 0:1] * enc_ref[0]
    for t in range(1, _TL):
        part = part + wblk[:, t:t + 1] * enc_ref[t]

    @pl.when(j == 0)
    def _():
        acc_ref[...] = part

    @pl.when(j > 0)
    def _():
        acc_ref[...] = acc_ref[...] + part


def _embed_attn_apply(input_ids, emb, h0, attn_W, attn_b2, enc):
    return pl.pallas_call(
        _embed_attn_apply_body,
        grid=(_NLB,),
        out_shape=[jax.ShapeDtypeStruct((_B, _H), _F32),
                   jax.ShapeDtypeStruct((_B, _L), _F32),
                   jax.ShapeDtypeStruct((_B, _H), _F32)],
        in_specs=[pl.BlockSpec(memory_space=pltpu.SMEM),
                  pl.BlockSpec(memory_space=pl.ANY),
                  pl.BlockSpec((_B, _H), lambda j: (0, 0)),
                  pl.BlockSpec((_L, 2 * _H), lambda j: (0, 0)),
                  pl.BlockSpec((1, _L), lambda j: (0, 0)),
                  pl.BlockSpec((_TL, _B, _H), lambda j: (j, 0, 0))],
        out_specs=[pl.BlockSpec((_B, _H), lambda j: (0, 0)),
                   pl.BlockSpec((_B, _L), lambda j: (0, 0)),
                   pl.BlockSpec((_B, _H), lambda j: (0, 0))],
        scratch_shapes=[pltpu.VMEM((_L, _B), _F32), pltpu.SemaphoreType.DMA],
        compiler_params=pltpu.CompilerParams(
            dimension_semantics=("arbitrary",),
            vmem_limit_bytes=48 * 1024 * 1024),
        name="embed_attn_apply",
    )(input_ids, emb, h0, attn_W, attn_b2, enc)


def _gru_body(emb_ref, aa_ref, h0_ref, cW_ref, cb_ref, Wih_ref, Whh_ref,
              bih_ref, bhh_ref, gW_ref, gb_ref, h_ref, pg_ref):
    h0 = h0_ref[...]
    combined = (_dot_nt(emb_ref[...], cW_ref[:, :_H])
                + _dot_nt(aa_ref[...], cW_ref[:, _H:]) + cb_ref[...])
    gi = _dot_nt(combined, Wih_ref[...]) + bih_ref[...]
    gh = _dot_nt(h0, Whh_ref[...]) + bhh_ref[...]
    r = jax.nn.sigmoid(gi[:, :_H] + gh[:, :_H])
    z = jax.nn.sigmoid(gi[:, _H:2 * _H] + gh[:, _H:2 * _H])
    n = jnp.tanh(gi[:, 2 * _H:] + r * gh[:, 2 * _H:])
    h_ref[...] = (1.0 - z) * n + z * h0
    g = (jnp.sum(combined * gW_ref[:, :_H], axis=1, keepdims=True)
         + jnp.sum(h0 * gW_ref[:, _H:], axis=1, keepdims=True) + gb_ref[0, 0])
    pg_ref[...] = jnp.broadcast_to(jax.nn.sigmoid(g), (_B, 128))


def _gru(embedded, aa, h0, comb_W, cb2, W_ih, W_hh, bih2, bhh2, gen_W, gb2):
    full = lambda *s: pl.BlockSpec(s, lambda: tuple(0 for _ in s))
    return pl.pallas_call(
        _gru_body,
        out_shape=[jax.ShapeDtypeStruct((_B, _H), _F32),
                   jax.ShapeDtypeStruct((_B, 128), _F32)],
        in_specs=[full(_B, _H), full(_B, _H), full(_B, _H),
                  full(_H, 2 * _H), full(1, _H),
                  full(3 * _H, _H), full(3 * _H, _H),
                  full(1, 3 * _H), full(1, 3 * _H),
                  full(1, 2 * _H), full(1, 128)],
        out_specs=[full(_B, _H), full(_B, 128)],
        compiler_params=pltpu.CompilerParams(vmem_limit_bytes=52 * 1024 * 1024),
        name="gru_step",
    )(embedded, aa, h0, comb_W, cb2, W_ih, W_hh, bih2, bhh2, gen_W, gb2)


def _logits_body(h_ref, W_ref, b_ref, out_ref, lse_ref, m_ref, s_ref):
    i = pl.program_id(0)
    res = _dot_nt(h_ref[...], W_ref[...]) + b_ref[0]
    for k in range(_RPS):
        out_ref[:, k, :] = res[:, k * _NLO:(k + 1) * _NLO].astype(jnp.bfloat16)

    col = jax.lax.broadcasted_iota(jnp.int32, (_B, _TV), 1)
    resm = jnp.where(col < _V - i * _TV, res, -1e30)
    bm = jnp.max(resm, axis=1, keepdims=True)

    @pl.when(i == 0)
    def _():
        m_ref[...] = jnp.broadcast_to(bm, (_B, 128))
        s_ref[...] = jnp.broadcast_to(
            jnp.sum(jnp.exp(resm - bm), axis=1, keepdims=True), (_B, 128))

    @pl.when(i > 0)
    def _():
        m_old = m_ref[:, 0:1]
        m_new = jnp.maximum(m_old, bm)
        s_new = (s_ref[:, 0:1] * jnp.exp(m_old - m_new)
                 + jnp.sum(jnp.exp(resm - m_new), axis=1, keepdims=True))
        m_ref[...] = jnp.broadcast_to(m_new, (_B, 128))
        s_ref[...] = jnp.broadcast_to(s_new, (_B, 128))

    @pl.when(i == _NV - 1)
    def _():
        lse_ref[...] = m_ref[...] + jnp.log(s_ref[...])


def _logits(h_new, out_W, out_b_pad):
    return pl.pallas_call(
        _logits_body,
        grid=(_NV,),
        out_shape=[jax.ShapeDtypeStruct((_B, _NHI, _NLO), jnp.bfloat16),
                   jax.ShapeDtypeStruct((_B, 128), _F32)],
        in_specs=[pl.BlockSpec((_B, _H), lambda i: (0, 0)),
                  pl.BlockSpec((_TV, _H), lambda i: (i, 0)),
                  pl.BlockSpec((1, 1, _TV), lambda i: (i, 0, 0))],
        out_specs=[pl.BlockSpec((_B, _RPS, _NLO), lambda i: (0, i, 0)),
                   pl.BlockSpec((_B, 128), lambda i: (0, 0))],
        scratch_shapes=[pltpu.VMEM((_B, 128), _F32),
                        pltpu.VMEM((_B, 128), _F32)],
        compiler_params=pltpu.CompilerParams(
            dimension_semantics=("arbitrary",),
            vmem_limit_bytes=50 * 1024 * 1024),
        name="vocab_logits",
    )(h_new, out_W, out_b_pad)


def _final_body(lg_ref, lse_ref, fr_ref, ar_ref, pg_ref, out_ref):
    r_iota = jax.lax.broadcasted_iota(jnp.int32, (_NHI, _NLO), 0)
    valid = r_iota < (_V // _NLO)
    for r in range(_RB):
        lg = lg_ref[r].astype(_F32)
        lse = lse_ref[r, 0, 0]
        idx_r = fr_ref[r]
        hi_r = (idx_r.astype(_F32) * (1.0 / _NLO)).astype(jnp.int32)
        lo_r = idx_r - hi_r * _NLO
        w_r = jnp.broadcast_to(ar_ref[r], (_NHI, _L))
        hi_eq = jax.lax.broadcasted_iota(jnp.int32, (_NHI, _L), 0) == hi_r
        hi_w = jnp.where(hi_eq, w_r, 0.0)
        lo_eq = jax.lax.broadcasted_iota(jnp.int32, (_NLO, _L), 0) == lo_r
        lo_t = jnp.where(lo_eq, 1.0, 0.0)
        td = _dot_nt(hi_w, lo_t)
        pg = pg_ref[r, 0, 0]
        out_ref[r] = jnp.where(valid, (lg - lse) * pg + (1.0 - pg) * td, 0.0)


def _finalize(logits_pad, lse, full_r, attn_r, pgen):
    return pl.pallas_call(
        _final_body,
        grid=(_B // _RB,),
        out_shape=jax.ShapeDtypeStruct((_B, _NHI, _NLO), _F32),
        in_specs=[pl.BlockSpec((_RB, _NHI, _NLO), lambda b: (b, 0, 0)),
                  pl.BlockSpec((_RB, 1, 128), lambda b: (b, 0, 0)),
                  pl.BlockSpec((_RB, 1, _L), lambda b: (b, 0, 0)),
                  pl.BlockSpec((_RB, 1, _L), lambda b: (b, 0, 0)),
                  pl.BlockSpec((_RB, 1, 128), lambda b: (b, 0, 0))],
        out_specs=pl.BlockSpec((_RB, _NHI, _NLO), lambda b: (b, 0, 0)),
        compiler_params=pltpu.CompilerParams(
            dimension_semantics=("arbitrary",),
            vmem_limit_bytes=32 * 1024 * 1024),
        name="scatter_final",
    )(logits_pad, lse, full_r, attn_r, pgen)


def kernel(input_ids, hidden, encoder_outputs, full_input, emb, attn_W,
           attn_b, comb_W, comb_b, W_ih, W_hh, b_ih, b_hh, out_W, out_b,
           gen_W, gen_b):
    h0 = hidden[0]
    embedded, attn_w, aa = _embed_attn_apply(
        input_ids, emb, h0, attn_W, attn_b.reshape(1, _L), encoder_outputs)

    h_new, pgen = _gru(embedded, aa, h0, comb_W, comb_b.reshape(1, _H),
                       W_ih, W_hh, b_ih.reshape(1, 3 * _H),
                       b_hh.reshape(1, 3 * _H), gen_W,
                       jnp.broadcast_to(gen_b.reshape(1, 1), (1, 128)))

    out_b_pad = jnp.pad(out_b, (0, _NV * _TV - _V)).reshape(_NV, 1, _TV)
    logits_pad, lse = _logits(h_new, out_W, out_b_pad)

    p_pad = _finalize(logits_pad, lse.reshape(_B, 1, 128),
                      full_input.reshape(_B, 1, _L),
                      attn_w.reshape(_B, 1, _L), pgen.reshape(_B, 1, 128))

    p_final = p_pad.reshape(_B, _WPAD)
    return p_final, hidden, attn_w


# speedup vs baseline: 1.0120x; 1.0120x over previous
import jax
import jax.numpy as jnp
from jax.experimental import pallas as pl
from jax.experimental.pallas import tpu as pltpu

_V = 50000
_H = 1024
_L = 400
_B = 64
_PAD = 500
_NHI = 101
_NLO = 500
_WPAD = _NHI * _NLO
_TV = 4000
_NV = 13
_RPS = _TV // _NLO
_TL = 40
_NLB = 10
_RB = 2
_F32 = jnp.float32


def _dot_nt(x, w):
    return jax.lax.dot_general(
        x, w, (((1,), (1,)), ((), ())), preferred_element_type=_F32)


def _embed_attn_apply_body(ids_ref, emb_hbm, h0_ref, aW_ref, ab_ref, enc_ref,
                           eout_ref, w_ref, acc_ref, wT_ref, sem):
    j = pl.program_id(0)

    @pl.when(j == 0)
    def _():
        cps = [pltpu.make_async_copy(emb_hbm.at[pl.ds(ids_ref[i], 1), :],
                                     eout_ref.at[pl.ds(i, 1), :], sem)
               for i in range(_B)]
        for cp in cps:
            cp.start()
        for cp in cps:
            cp.wait()
        emb = eout_ref[...]
        logits = (_dot_nt(emb, aW_ref[:, :_H])
                  + _dot_nt(h0_ref[...], aW_ref[:, _H:]) + ab_ref[...])
        m = jnp.max(logits, axis=1, keepdims=True)
        e = jnp.exp(logits - m)
        w = e / jnp.sum(e, axis=1, keepdims=True)
        w_ref[...] = w
        wT_ref[...] = w.T

    off = pl.multiple_of(j * _TL, 8)
    wblk = jnp.transpose(wT_ref[pl.ds(off, _TL), :])
    part = wblk[:, 0:1] * enc_ref[0]
    for t in range(1, _TL):
        part = part + wblk[:, t:t + 1] * enc_ref[t]

    @pl.when(j == 0)
    def _():
        acc_ref[...] = part

    @pl.when(j > 0)
    def _():
        acc_ref[...] = acc_ref[...] + part


def _embed_attn_apply(input_ids, emb, h0, attn_W, attn_b2, enc):
    return pl.pallas_call(
        _embed_attn_apply_body,
        grid=(_NLB,),
        out_shape=[jax.ShapeDtypeStruct((_B, _H), _F32),
                   jax.ShapeDtypeStruct((_B, _L), _F32),
                   jax.ShapeDtypeStruct((_B, _H), _F32)],
        in_specs=[pl.BlockSpec(memory_space=pltpu.SMEM),
                  pl.BlockSpec(memory_space=pl.ANY),
                  pl.BlockSpec((_B, _H), lambda j: (0, 0)),
                  pl.BlockSpec((_L, 2 * _H), lambda j: (0, 0)),
                  pl.BlockSpec((1, _L), lambda j: (0, 0)),
                  pl.BlockSpec((_TL, _B, _H), lambda j: (j, 0, 0))],
        out_specs=[pl.BlockSpec((_B, _H), lambda j: (0, 0)),
                   pl.BlockSpec((_B, _L), lambda j: (0, 0)),
                   pl.BlockSpec((_B, _H), lambda j: (0, 0))],
        scratch_shapes=[pltpu.VMEM((_L, _B), _F32), pltpu.SemaphoreType.DMA],
        compiler_params=pltpu.CompilerParams(
            dimension_semantics=("arbitrary",),
            vmem_limit_bytes=48 * 1024 * 1024),
        name="embed_attn_apply",
    )(input_ids, emb, h0, attn_W, attn_b2, enc)


def _gru_body(emb_ref, aa_ref, h0_ref, cW_ref, cb_ref, Wih_ref, Whh_ref,
              bih_ref, bhh_ref, gW_ref, gb_ref, h_ref, pg_ref):
    h0 = h0_ref[...]
    combined = (_dot_nt(emb_ref[...], cW_ref[:, :_H])
                + _dot_nt(aa_ref[...], cW_ref[:, _H:]) + cb_ref[...])
    gi = _dot_nt(combined, Wih_ref[...]) + bih_ref[...]
    gh = _dot_nt(h0, Whh_ref[...]) + bhh_ref[...]
    r = jax.nn.sigmoid(gi[:, :_H] + gh[:, :_H])
    z = jax.nn.sigmoid(gi[:, _H:2 * _H] + gh[:, _H:2 * _H])
    n = jnp.tanh(gi[:, 2 * _H:] + r * gh[:, 2 * _H:])
    h_ref[...] = (1.0 - z) * n + z * h0
    g = (jnp.sum(combined * gW_ref[:, :_H], axis=1, keepdims=True)
         + jnp.sum(h0 * gW_ref[:, _H:], axis=1, keepdims=True) + gb_ref[0, 0])
    pg_ref[...] = jnp.broadcast_to(jax.nn.sigmoid(g), (_B, 128))


def _gru(embedded, aa, h0, comb_W, cb2, W_ih, W_hh, bih2, bhh2, gen_W, gb2):
    full = lambda *s: pl.BlockSpec(s, lambda: tuple(0 for _ in s))
    return pl.pallas_call(
        _gru_body,
        out_shape=[jax.ShapeDtypeStruct((_B, _H), _F32),
                   jax.ShapeDtypeStruct((_B, 128), _F32)],
        in_specs=[full(_B, _H), full(_B, _H), full(_B, _H),
                  full(_H, 2 * _H), full(1, _H),
                  full(3 * _H, _H), full(3 * _H, _H),
                  full(1, 3 * _H), full(1, 3 * _H),
                  full(1, 2 * _H), full(1, 128)],
        out_specs=[full(_B, _H), full(_B, 128)],
        compiler_params=pltpu.CompilerParams(vmem_limit_bytes=52 * 1024 * 1024),
        name="gru_step",
    )(embedded, aa, h0, comb_W, cb2, W_ih, W_hh, bih2, bhh2, gen_W, gb2)


def _logits_body(h_ref, W_ref, b_ref, out_ref, lse_ref, m_ref, s_ref):
    i = pl.program_id(0)
    res = _dot_nt(h_ref[...], W_ref[...]) + b_ref[0]
    for k in range(_RPS):
        out_ref[:, k, :] = res[:, k * _NLO:(k + 1) * _NLO].astype(jnp.bfloat16)

    col = jax.lax.broadcasted_iota(jnp.int32, (_B, _TV), 1)
    resm = jnp.where(col < _V - i * _TV, res, -1e30)
    bm = jnp.max(resm, axis=1, keepdims=True)

    @pl.when(i == 0)
    def _():
        m_ref[...] = jnp.broadcast_to(bm, (_B, 128))
        s_ref[...] = jnp.broadcast_to(
            jnp.sum(jnp.exp(resm - bm), axis=1, keepdims=True), (_B, 128))

    @pl.when(i > 0)
    def _():
        m_old = m_ref[:, 0:1]
        m_new = jnp.maximum(m_old, bm)
        s_new = (s_ref[:, 0:1] * jnp.exp(m_old - m_new)
                 + jnp.sum(jnp.exp(resm - m_new), axis=1, keepdims=True))
        m_ref[...] = jnp.broadcast_to(m_new, (_B, 128))
        s_ref[...] = jnp.broadcast_to(s_new, (_B, 128))

    @pl.when(i == _NV - 1)
    def _():
        lse_ref[...] = m_ref[...] + jnp.log(s_ref[...])


def _logits(h_new, out_W, out_b_pad):
    return pl.pallas_call(
        _logits_body,
        grid=(_NV,),
        out_shape=[jax.ShapeDtypeStruct((_B, _NHI, _NLO), jnp.bfloat16),
                   jax.ShapeDtypeStruct((_B, 128), _F32)],
        in_specs=[pl.BlockSpec((_B, _H), lambda i: (0, 0)),
                  pl.BlockSpec((_TV, _H), lambda i: (i, 0)),
                  pl.BlockSpec((1, 1, _TV), lambda i: (i, 0, 0))],
        out_specs=[pl.BlockSpec((_B, _RPS, _NLO), lambda i: (0, i, 0)),
                   pl.BlockSpec((_B, 128), lambda i: (0, 0))],
        scratch_shapes=[pltpu.VMEM((_B, 128), _F32),
                        pltpu.VMEM((_B, 128), _F32)],
        compiler_params=pltpu.CompilerParams(
            dimension_semantics=("arbitrary",),
            vmem_limit_bytes=50 * 1024 * 1024),
        name="vocab_logits",
    )(h_new, out_W, out_b_pad)


def _final_body(lg_ref, lse_ref, fr_ref, ar_ref, pg_ref, out_ref):
    r_iota = jax.lax.broadcasted_iota(jnp.int32, (_NHI, _NLO), 0)
    valid = r_iota < (_V // _NLO)
    for r in range(_RB):
        lg = lg_ref[r].astype(_F32)
        lse = lse_ref[r, 0, 0]
        idx_r = fr_ref[r]
        hi_r = (idx_r.astype(_F32) * (1.0 / _NLO)).astype(jnp.int32)
        lo_r = idx_r - hi_r * _NLO
        w_r = jnp.broadcast_to(ar_ref[r], (_NHI, _L))
        hi_eq = jax.lax.broadcasted_iota(jnp.int32, (_NHI, _L), 0) == hi_r
        hi_w = jnp.where(hi_eq, w_r, 0.0)
        lo_eq = jax.lax.broadcasted_iota(jnp.int32, (_NLO, _L), 0) == lo_r
        lo_t = jnp.where(lo_eq, 1.0, 0.0)
        td = _dot_nt(hi_w, lo_t)
        pg = pg_ref[r, 0, 0]
        out_ref[r] = jnp.where(valid, (lg - lse) * pg + (1.0 - pg) * td, 0.0)


def _finalize(logits_pad, lse, full_r, attn_r, pgen):
    return pl.pallas_call(
        _final_body,
        grid=(_B // _RB,),
        out_shape=jax.ShapeDtypeStruct((_B, _NHI, _NLO), _F32),
        in_specs=[pl.BlockSpec((_RB, _NHI, _NLO), lambda b: (b, 0, 0)),
                  pl.BlockSpec((_RB, 1, 128), lambda b: (b, 0, 0)),
                  pl.BlockSpec((_RB, 1, _L), lambda b: (b, 0, 0)),
                  pl.BlockSpec((_RB, 1, _L), lambda b: (b, 0, 0)),
                  pl.BlockSpec((_RB, 1, 128), lambda b: (b, 0, 0))],
        out_specs=pl.BlockSpec((_RB, _NHI, _NLO), lambda b: (b, 0, 0)),
        compiler_params=pltpu.CompilerParams(
            dimension_semantics=("arbitrary",),
            vmem_limit_bytes=32 * 1024 * 1024),
        name="scatter_final",
    )(logits_pad, lse, full_r, attn_r, pgen)


def kernel(input_ids, hidden, encoder_outputs, full_input, emb, attn_W,
           attn_b, comb_W, comb_b, W_ih, W_hh, b_ih, b_hh, out_W, out_b,
           gen_W, gen_b):
    h0 = hidden[0]
    embedded, attn_w, aa = _embed_attn_apply(
        input_ids, emb, h0, attn_W, attn_b.reshape(1, _L), encoder_outputs)

    h_new, pgen = _gru(embedded, aa, h0, comb_W, comb_b.reshape(1, _H),
                       W_ih, W_hh, b_ih.reshape(1, 3 * _H),
                       b_hh.reshape(1, 3 * _H), gen_W,
                       jnp.broadcast_to(gen_b.reshape(1, 1), (1, 128)))

    out_b_pad = jnp.pad(out_b, (0, _NV * _TV - _V)).reshape(_NV, 1, _TV)
    logits_pad, lse = _logits(h_new, out_W, out_b_pad)

    p_pad = _finalize(logits_pad, lse.reshape(_B, 1, 128),
                      full_input.reshape(_B, 1, _L),
                      attn_w.reshape(_B, 1, _L), pgen.reshape(_B, 1, 128))

    p_final = p_pad.reshape(_B, _WPAD)
    return p_final, hidden, attn_w


# speedup vs baseline: 1.0297x; 1.0174x over previous
import jax
import jax.numpy as jnp
from jax.experimental import pallas as pl
from jax.experimental.pallas import tpu as pltpu

_V = 50000
_H = 1024
_L = 400
_B = 64
_PAD = 500
_NHI = 101
_NLO = 500
_WPAD = _NHI * _NLO
_TV = 4000
_NV = 13
_RPS = _TV // _NLO
_TL = 40
_NLB = 10
_RB = 4
_F32 = jnp.float32


def _dot_nt(x, w):
    return jax.lax.dot_general(
        x, w, (((1,), (1,)), ((), ())), preferred_element_type=_F32)


def _embed_attn_apply_body(ids_ref, emb_hbm, h0_ref, aW_ref, ab_ref, enc_ref,
                           eout_ref, w_ref, acc_ref, wT_ref, sem):
    j = pl.program_id(0)

    @pl.when(j == 0)
    def _():
        cps = [pltpu.make_async_copy(emb_hbm.at[pl.ds(ids_ref[i], 1), :],
                                     eout_ref.at[pl.ds(i, 1), :], sem)
               for i in range(_B)]
        for cp in cps:
            cp.start()
        for cp in cps:
            cp.wait()
        emb = eout_ref[...]
        logits = (_dot_nt(emb, aW_ref[:, :_H])
                  + _dot_nt(h0_ref[...], aW_ref[:, _H:]) + ab_ref[...])
        m = jnp.max(logits, axis=1, keepdims=True)
        e = jnp.exp(logits - m)
        w = e / jnp.sum(e, axis=1, keepdims=True)
        w_ref[...] = w
        wT_ref[...] = w.T

    off = pl.multiple_of(j * _TL, 8)
    wblk = jnp.transpose(wT_ref[pl.ds(off, _TL), :])
    part = wblk[:, 0:1] * enc_ref[0]
    for t in range(1, _TL):
        part = part + wblk[:, t:t + 1] * enc_ref[t]

    @pl.when(j == 0)
    def _():
        acc_ref[...] = part

    @pl.when(j > 0)
    def _():
        acc_ref[...] = acc_ref[...] + part


def _embed_attn_apply(input_ids, emb, h0, attn_W, attn_b2, enc):
    return pl.pallas_call(
        _embed_attn_apply_body,
        grid=(_NLB,),
        out_shape=[jax.ShapeDtypeStruct((_B, _H), _F32),
                   jax.ShapeDtypeStruct((_B, _L), _F32),
                   jax.ShapeDtypeStruct((_B, _H), _F32)],
        in_specs=[pl.BlockSpec(memory_space=pltpu.SMEM),
                  pl.BlockSpec(memory_space=pl.ANY),
                  pl.BlockSpec((_B, _H), lambda j: (0, 0)),
                  pl.BlockSpec((_L, 2 * _H), lambda j: (0, 0)),
                  pl.BlockSpec((1, _L), lambda j: (0, 0)),
                  pl.BlockSpec((_TL, _B, _H), lambda j: (j, 0, 0))],
        out_specs=[pl.BlockSpec((_B, _H), lambda j: (0, 0)),
                   pl.BlockSpec((_B, _L), lambda j: (0, 0)),
                   pl.BlockSpec((_B, _H), lambda j: (0, 0))],
        scratch_shapes=[pltpu.VMEM((_L, _B), _F32), pltpu.SemaphoreType.DMA],
        compiler_params=pltpu.CompilerParams(
            dimension_semantics=("arbitrary",),
            vmem_limit_bytes=48 * 1024 * 1024),
        name="embed_attn_apply",
    )(input_ids, emb, h0, attn_W, attn_b2, enc)


def _gru_body(emb_ref, aa_ref, h0_ref, cW_ref, cb_ref, Wih_ref, Whh_ref,
              bih_ref, bhh_ref, gW_ref, gb_ref, h_ref, pg_ref):
    h0 = h0_ref[...]
    combined = (_dot_nt(emb_ref[...], cW_ref[:, :_H])
                + _dot_nt(aa_ref[...], cW_ref[:, _H:]) + cb_ref[...])
    gi = _dot_nt(combined, Wih_ref[...]) + bih_ref[...]
    gh = _dot_nt(h0, Whh_ref[...]) + bhh_ref[...]
    r = jax.nn.sigmoid(gi[:, :_H] + gh[:, :_H])
    z = jax.nn.sigmoid(gi[:, _H:2 * _H] + gh[:, _H:2 * _H])
    n = jnp.tanh(gi[:, 2 * _H:] + r * gh[:, 2 * _H:])
    h_ref[...] = (1.0 - z) * n + z * h0
    g = (jnp.sum(combined * gW_ref[:, :_H], axis=1, keepdims=True)
         + jnp.sum(h0 * gW_ref[:, _H:], axis=1, keepdims=True) + gb_ref[0, 0])
    pg_ref[...] = jnp.broadcast_to(jax.nn.sigmoid(g), (_B, 128))


def _gru(embedded, aa, h0, comb_W, cb2, W_ih, W_hh, bih2, bhh2, gen_W, gb2):
    full = lambda *s: pl.BlockSpec(s, lambda: tuple(0 for _ in s))
    return pl.pallas_call(
        _gru_body,
        out_shape=[jax.ShapeDtypeStruct((_B, _H), _F32),
                   jax.ShapeDtypeStruct((_B, 128), _F32)],
        in_specs=[full(_B, _H), full(_B, _H), full(_B, _H),
                  full(_H, 2 * _H), full(1, _H),
                  full(3 * _H, _H), full(3 * _H, _H),
                  full(1, 3 * _H), full(1, 3 * _H),
                  full(1, 2 * _H), full(1, 128)],
        out_specs=[full(_B, _H), full(_B, 128)],
        compiler_params=pltpu.CompilerParams(vmem_limit_bytes=52 * 1024 * 1024),
        name="gru_step",
    )(embedded, aa, h0, comb_W, cb2, W_ih, W_hh, bih2, bhh2, gen_W, gb2)


def _logits_body(h_ref, W_ref, b_ref, out_ref, lse_ref, m_ref, s_ref):
    i = pl.program_id(0)
    res = _dot_nt(h_ref[...], W_ref[...]) + b_ref[0]
    for k in range(_RPS):
        out_ref[:, k, :] = res[:, k * _NLO:(k + 1) * _NLO].astype(jnp.bfloat16)

    col = jax.lax.broadcasted_iota(jnp.int32, (_B, _TV), 1)
    resm = jnp.where(col < _V - i * _TV, res, -1e30)
    bm = jnp.max(resm, axis=1, keepdims=True)

    @pl.when(i == 0)
    def _():
        m_ref[...] = jnp.broadcast_to(bm, (_B, 128))
        s_ref[...] = jnp.broadcast_to(
            jnp.sum(jnp.exp(resm - bm), axis=1, keepdims=True), (_B, 128))

    @pl.when(i > 0)
    def _():
        m_old = m_ref[:, 0:1]
        m_new = jnp.maximum(m_old, bm)
        s_new = (s_ref[:, 0:1] * jnp.exp(m_old - m_new)
                 + jnp.sum(jnp.exp(resm - m_new), axis=1, keepdims=True))
        m_ref[...] = jnp.broadcast_to(m_new, (_B, 128))
        s_ref[...] = jnp.broadcast_to(s_new, (_B, 128))

    @pl.when(i == _NV - 1)
    def _():
        lse_ref[...] = m_ref[...] + jnp.log(s_ref[...])


def _logits(h_new, out_W, out_b_pad):
    return pl.pallas_call(
        _logits_body,
        grid=(_NV,),
        out_shape=[jax.ShapeDtypeStruct((_B, _NHI, _NLO), jnp.bfloat16),
                   jax.ShapeDtypeStruct((_B, 128), _F32)],
        in_specs=[pl.BlockSpec((_B, _H), lambda i: (0, 0)),
                  pl.BlockSpec((_TV, _H), lambda i: (i, 0)),
                  pl.BlockSpec((1, 1, _TV), lambda i: (i, 0, 0))],
        out_specs=[pl.BlockSpec((_B, _RPS, _NLO), lambda i: (0, i, 0)),
                   pl.BlockSpec((_B, 128), lambda i: (0, 0))],
        scratch_shapes=[pltpu.VMEM((_B, 128), _F32),
                        pltpu.VMEM((_B, 128), _F32)],
        compiler_params=pltpu.CompilerParams(
            dimension_semantics=("arbitrary",),
            vmem_limit_bytes=50 * 1024 * 1024),
        name="vocab_logits",
    )(h_new, out_W, out_b_pad)


def _final_body(lg_ref, lse_ref, fr_ref, ar_ref, pg_ref, out_ref):
    r_iota = jax.lax.broadcasted_iota(jnp.int32, (_NHI, _NLO), 0)
    valid = r_iota < (_V // _NLO)
    for r in range(_RB):
        lg = lg_ref[r].astype(_F32)
        lse = lse_ref[r, 0, 0]
        idx_r = fr_ref[r]
        hi_r = (idx_r.astype(_F32) * (1.0 / _NLO)).astype(jnp.int32)
        lo_r = idx_r - hi_r * _NLO
        w_r = jnp.broadcast_to(ar_ref[r], (_NHI, _L))
        hi_eq = jax.lax.broadcasted_iota(jnp.int32, (_NHI, _L), 0) == hi_r
        hi_w = jnp.where(hi_eq, w_r, 0.0)
        lo_eq = jax.lax.broadcasted_iota(jnp.int32, (_NLO, _L), 0) == lo_r
        lo_t = jnp.where(lo_eq, 1.0, 0.0)
        td = _dot_nt(hi_w, lo_t)
        pg = pg_ref[r, 0, 0]
        out_ref[r] = jnp.where(valid, (lg - lse) * pg + (1.0 - pg) * td, 0.0)


def _finalize(logits_pad, lse, full_r, attn_r, pgen):
    return pl.pallas_call(
        _final_body,
        grid=(_B // _RB,),
        out_shape=jax.ShapeDtypeStruct((_B, _NHI, _NLO), _F32),
        in_specs=[pl.BlockSpec((_RB, _NHI, _NLO), lambda b: (b, 0, 0)),
                  pl.BlockSpec((_RB, 1, 128), lambda b: (b, 0, 0)),
                  pl.BlockSpec((_RB, 1, _L), lambda b: (b, 0, 0)),
                  pl.BlockSpec((_RB, 1, _L), lambda b: (b, 0, 0)),
                  pl.BlockSpec((_RB, 1, 128), lambda b: (b, 0, 0))],
        out_specs=pl.BlockSpec((_RB, _NHI, _NLO), lambda b: (b, 0, 0)),
        compiler_params=pltpu.CompilerParams(
            dimension_semantics=("arbitrary",),
            vmem_limit_bytes=32 * 1024 * 1024),
        name="scatter_final",
    )(logits_pad, lse, full_r, attn_r, pgen)


def kernel(input_ids, hidden, encoder_outputs, full_input, emb, attn_W,
           attn_b, comb_W, comb_b, W_ih, W_hh, b_ih, b_hh, out_W, out_b,
           gen_W, gen_b):
    h0 = hidden[0]
    embedded, attn_w, aa = _embed_attn_apply(
        input_ids, emb, h0, attn_W, attn_b.reshape(1, _L), encoder_outputs)

    h_new, pgen = _gru(embedded, aa, h0, comb_W, comb_b.reshape(1, _H),
                       W_ih, W_hh, b_ih.reshape(1, 3 * _H),
                       b_hh.reshape(1, 3 * _H), gen_W,
                       jnp.broadcast_to(gen_b.reshape(1, 1), (1, 128)))

    out_b_pad = jnp.pad(out_b, (0, _NV * _TV - _V)).reshape(_NV, 1, _TV)
    logits_pad, lse = _logits(h_new, out_W, out_b_pad)

    p_pad = _finalize(logits_pad, lse.reshape(_B, 1, 128),
                      full_input.reshape(_B, 1, _L),
                      attn_w.reshape(_B, 1, _L), pgen.reshape(_B, 1, 128))

    p_final = p_pad.reshape(_B, _WPAD)
    return p_final, hidden, attn_w
